# baseline (speedup 1.0000x reference)
import functools

import jax
import jax.numpy as jnp
from jax import lax
from jax.experimental import pallas as pl
from jax.experimental.pallas import tpu as pltpu
from jax.experimental.pallas import tpu_sc as plsc

N = 10000
NPAD = 10240
NH = NPAD // 2
NB = 512
NBLK = NPAD // NB
D = 256
H = 128
G = 64

E = 160000
NC = 2
NS = 16
NT = NC * NS
TILE_E = 10240
EPAD = TILE_E * NS
ROWS_E = EPAD // 128
NCHUNK_E = TILE_E // 128
TILE_D = 5120
EPAD_D = TILE_D * NT
ROWS_D = EPAD_D // 128
NCHUNK_D = TILE_D // 128
NROWS_T = NH // NS

_mesh = plsc.VectorSubcoreMesh(core_axis_name="c", subcore_axis_name="s")
_sc_params = pltpu.CompilerParams(needs_layout_passes=False)


@functools.partial(
    pl.kernel,
    out_type=jax.ShapeDtypeStruct((NT, NPAD), jnp.float32),
    scratch_types=[
        pltpu.VMEM((NCHUNK_D, 128), jnp.int32),
        pltpu.VMEM((NPAD,), jnp.float32),
        pltpu.SemaphoreType.DMA,
    ],
    mesh=_mesh,
    compiler_params=_sc_params,
)
def _deg_kernel(dst_h, out_h, idx_v, deg_v, sem):
    c = lax.axis_index("c")
    s = lax.axis_index("s")
    tile = s * NC + c
    zero = jnp.zeros((16,), jnp.float32)

    def fill(i, _):
        deg_v[pl.ds(i * 16, 16)] = zero
        return 0

    lax.fori_loop(0, NPAD // 16, fill, 0)
    pltpu.sync_copy(dst_h.at[pl.ds(tile * NCHUNK_D, NCHUNK_D)], idx_v)
    ones = jnp.full((16,), 1.0, jnp.float32)

    def chunk(j, _):
        for k in range(8):
            iv = idx_v[j, pl.ds(k * 16, 16)]
            plsc.addupdate_scatter(deg_v, [iv], ones)
        return 0

    lax.fori_loop(0, NCHUNK_D, chunk, 0)
    pltpu.sync_copy(deg_v, out_h.at[tile])


def _make_scatter(p):
    @functools.partial(
        pl.kernel,
        out_type=jax.ShapeDtypeStruct((2 * NH, 128), jnp.float32),
        scratch_types=[
            pltpu.VMEM((NCHUNK_E, 128), jnp.int32),
            pltpu.VMEM((NCHUNK_E, 128), jnp.int32),
            pltpu.VMEM((3, 128, 128), jnp.float32),
            pltpu.VMEM_SHARED((NH, 128), jnp.float32),
            pltpu.SemaphoreType.DMA((3,)),
        ],
        mesh=_mesh,
        compiler_params=_sc_params,
    )
    def _scatter(hp_h, src_h, dst_h, out_h, idxs_v, idxd_v, rows_v, acc_s,
                 gsem):
        c = lax.axis_index("c")
        s = lax.axis_index("s")
        base = s * NROWS_T
        pltpu.sync_copy(hp_h.at[pl.ds(c * NPAD + p * NH + base, NROWS_T)],
                        acc_s.at[pl.ds(base, NROWS_T)])
        plsc.subcore_barrier()

        pltpu.sync_copy(src_h.at[pl.ds(c * ROWS_E + s * NCHUNK_E, NCHUNK_E)],
                        idxs_v)
        pltpu.sync_copy(dst_h.at[pl.ds(s * NCHUNK_E, NCHUNK_E)], idxd_v)

        def gather(j):
            b = lax.rem(j, 3)
            return pltpu.make_async_copy(
                hp_h.at[plsc.Indices(idxs_v.at[j], ignored_value=-1)],
                rows_v.at[b], gsem.at[b])

        gather(jnp.int32(0)).start()
        gather(jnp.int32(1)).start()

        def chunk(j, _):
            b = lax.rem(j, 3)
            gather(j).wait()

            @pl.when(j < NCHUNK_E - 2)
            def _():
                gather(j + 2).start()

            pltpu.sync_copy(
                rows_v.at[b],
                acc_s.at[plsc.Indices(idxd_v.at[j], ignored_value=-1)],
                add=True)
            return 0

        lax.fori_loop(0, NCHUNK_E, chunk, 0)
        plsc.subcore_barrier()
        pltpu.sync_copy(acc_s.at[pl.ds(base, NROWS_T)],
                        out_h.at[pl.ds(c * NH + base, NROWS_T)])

    return _scatter


_scatter_lo = _make_scatter(0)
_scatter_hi = _make_scatter(1)


def _dinv128(degp_blk):
    deg = lax.dot_general(
        degp_blk, jnp.ones((NT, 128), jnp.float32),
        dimension_numbers=(((0,), (0,)), ((), ())),
        preferred_element_type=jnp.float32) + 1.0
    return lax.rsqrt(deg)


def _split_scaled(h, dinv):
    return jnp.stack([h[:, :128] * dinv, h[:, 128:] * dinv])


def _pre_body(x_ref, w_ref, degp_ref, hp_ref):
    dinv = _dinv128(degp_ref[...])
    h = jnp.dot(x_ref[...], w_ref[...], preferred_element_type=jnp.float32)
    hp_ref[...] = _split_scaled(h, dinv)


def _pre_call(xp, w1t, degp):
    return pl.pallas_call(
        _pre_body,
        grid=(NBLK,),
        in_specs=[
            pl.BlockSpec((NB, D), lambda nb: (nb, 0)),
            pl.BlockSpec((D, D), lambda nb: (0, 0)),
            pl.BlockSpec((NT, NB), lambda nb: (0, nb)),
        ],
        out_specs=pl.BlockSpec((2, NB, 128), lambda nb: (0, nb, 0)),
        out_shape=jax.ShapeDtypeStruct((2, NPAD, 128), jnp.float32),
    )(xp, w1t, degp)


def _seg_max(ho, bcol, lo, hi, p_ref):
    neg = jnp.float32(-jnp.inf)

    def seg(g, _):
        m = bcol == g.astype(jnp.float32)
        v = jnp.max(jnp.where(m, ho, neg), axis=0, keepdims=True)
        p_ref[pl.ds(g, 1), :] = jnp.maximum(p_ref[pl.ds(g, 1), :], v)
        return 0

    lax.fori_loop(lo, jnp.minimum(hi, G - 1) + 1, seg, 0)


def _update_body(acc_ref, degp_ref, b_ref, w_ref, bcol_ref, lo_ref, hi_ref,
                 hp_ref, p_ref):
    nb = pl.program_id(0)

    @pl.when(nb == 0)
    def _():
        p_ref[...] = jnp.full((G, D), -jnp.inf, jnp.float32)

    dinv = _dinv128(degp_ref[...])
    ho = jnp.concatenate([acc_ref[0] * dinv, acc_ref[1] * dinv], axis=-1)
    ho = jnp.maximum(ho + b_ref[...], 0.0)
    h = jnp.dot(ho, w_ref[...], preferred_element_type=jnp.float32)
    hp_ref[...] = _split_scaled(h, dinv)
    _seg_max(ho, bcol_ref[...], lo_ref[nb], hi_ref[nb], p_ref)


def _update_call(acc, degp, b, wt, bcol, lo, hi):
    return pl.pallas_call(
        _update_body,
        grid=(NBLK,),
        in_specs=[
            pl.BlockSpec((2, NB, 128), lambda nb: (0, nb, 0)),
            pl.BlockSpec((NT, NB), lambda nb: (0, nb)),
            pl.BlockSpec((1, D), lambda nb: (0, 0)),
            pl.BlockSpec((D, D), lambda nb: (0, 0)),
            pl.BlockSpec((NB, 1), lambda nb: (nb, 0)),
            pl.BlockSpec(memory_space=pltpu.MemorySpace.SMEM),
            pl.BlockSpec(memory_space=pltpu.MemorySpace.SMEM),
        ],
        out_specs=[
            pl.BlockSpec((2, NB, 128), lambda nb: (0, nb, 0)),
            pl.BlockSpec((G, D), lambda nb: (0, 0)),
        ],
        out_shape=[
            jax.ShapeDtypeStruct((2, NPAD, 128), jnp.float32),
            jax.ShapeDtypeStruct((G, D), jnp.float32),
        ],
    )(acc, degp, b, wt, bcol, lo, hi)


def _pool_body(acc_ref, degp_ref, b_ref, bcol_ref, lo_ref, hi_ref, p_ref):
    nb = pl.program_id(0)

    @pl.when(nb == 0)
    def _():
        p_ref[...] = jnp.full((G, D), -jnp.inf, jnp.float32)

    dinv = _dinv128(degp_ref[...])
    ho3 = jnp.concatenate([acc_ref[0] * dinv, acc_ref[1] * dinv], axis=-1)
    ho3 = jnp.maximum(ho3 + b_ref[...], 0.0)
    _seg_max(ho3, bcol_ref[...], lo_ref[nb], hi_ref[nb], p_ref)


def _pool_call(acc3, degp, b3, bcol, lo, hi):
    return pl.pallas_call(
        _pool_body,
        grid=(NBLK,),
        in_specs=[
            pl.BlockSpec((2, NB, 128), lambda nb: (0, nb, 0)),
            pl.BlockSpec((NT, NB), lambda nb: (0, nb)),
            pl.BlockSpec((1, D), lambda nb: (0, 0)),
            pl.BlockSpec((NB, 1), lambda nb: (nb, 0)),
            pl.BlockSpec(memory_space=pltpu.MemorySpace.SMEM),
            pl.BlockSpec(memory_space=pltpu.MemorySpace.SMEM),
        ],
        out_specs=pl.BlockSpec((G, D), lambda nb: (0, 0)),
        out_shape=jax.ShapeDtypeStruct((G, D), jnp.float32),
    )(acc3, degp, b3, bcol, lo, hi)


def _ln(x, g, b):
    m = jnp.mean(x, axis=-1, keepdims=True)
    v = jnp.mean((x - m) ** 2, axis=-1, keepdims=True)
    return (x - m) * lax.rsqrt(v + 1e-5) * g + b


def _head_body(pooled_ref, pi_ref, wgt_ref, bg_ref, gg_ref, gb_ref, wpt_ref,
               bp_ref, pg_ref, pb_ref, wf1a_ref, wf1b_ref, bf1_ref, f1g_ref,
               f1b_ref, wf2t_ref, bf2_ref, f2g_ref, f2b_ref, out_ref):
    ge = jnp.dot(pooled_ref[...], wgt_ref[...],
                 preferred_element_type=jnp.float32) + bg_ref[...]
    ge = jnp.maximum(_ln(ge, gg_ref[...], gb_ref[...]), 0.0)
    pe = jnp.dot(pi_ref[...], wpt_ref[...],
                 preferred_element_type=jnp.float32) + bp_ref[...]
    pe = jnp.maximum(_ln(pe, pg_ref[...], pb_ref[...]), 0.0)
    z = (jnp.dot(ge, wf1a_ref[...], preferred_element_type=jnp.float32)
         + jnp.dot(pe, wf1b_ref[...], preferred_element_type=jnp.float32)
         + bf1_ref[...])
    z = _ln(z, f1g_ref[...], f1b_ref[...])
    z = jnp.dot(z, wf2t_ref[...], preferred_element_type=jnp.float32) \
        + bf2_ref[...]
    out_ref[...] = _ln(z, f2g_ref[...], f2b_ref[...])


def _head_call(pooled, pi, *args):
    return pl.pallas_call(
        _head_body,
        out_shape=jax.ShapeDtypeStruct((G, 10), jnp.float32),
    )(pooled, pi, *args)


def _layer_scatter(hp, src, dst, dst_ok):
    hp = hp.reshape(2 * NPAD, 128)
    accs = []
    for p, call in ((0, _scatter_lo), (1, _scatter_hi)):
        keep = dst_ok & (dst // NH == p)
        srcq = jnp.where(keep[None, :], src[None, :]
                         + jnp.array([[0], [NPAD]], jnp.int32), -1)
        dstq = jnp.where(keep, dst - p * NH, -1)
        out = call(hp, srcq.reshape(2 * ROWS_E, 128),
                   dstq.reshape(ROWS_E, 128))
        accs.append(out.reshape(2, NH, 128))
    return jnp.concatenate(accs, axis=1)


def kernel(x, edge_index, batch, pi, W1, b1, W2, b2, W3, b3, Wg, bg, gg, gb,
           Wp, bp, pg, pb, Wf1, bf1, f1g, f1b, Wf2, bf2, f2g, f2b):
    f32 = jnp.float32
    src = edge_index[0]
    dst = edge_index[1]

    srcp = jnp.concatenate([src, jnp.zeros((EPAD - E,), jnp.int32)])
    dstp = jnp.concatenate([dst, jnp.zeros((EPAD - E,), jnp.int32)])
    dst_ok = jnp.arange(EPAD, dtype=jnp.int32) < E
    dstd = jnp.concatenate([dst, jnp.full((EPAD_D - E,), N, jnp.int32)])

    xp = jnp.pad(x, ((0, NPAD - N), (0, 0)))
    batchp = jnp.concatenate([batch, jnp.full((NPAD - N,), G, jnp.int32)])
    bcol = batchp.astype(f32)[:, None]
    lo = batchp[::NB]
    hi = batchp[NB - 1::NB]

    degp = _deg_kernel(dstd.reshape(ROWS_D, 128))

    hp1 = _pre_call(xp, W1.T, degp)
    acc1 = _layer_scatter(hp1, srcp, dstp, dst_ok)
    hp2, p1 = _update_call(acc1, degp, b1[None, :], W2.T, bcol, lo, hi)
    acc2 = _layer_scatter(hp2, srcp, dstp, dst_ok)
    hp3, p2 = _update_call(acc2, degp, b2[None, :], W3.T, bcol, lo, hi)
    acc3 = _layer_scatter(hp3, srcp, dstp, dst_ok)

    p3 = _pool_call(acc3, degp, b3[None, :], bcol, lo, hi)
    pooled = jnp.concatenate([p1, p2, p3], axis=1)

    return _head_call(
        pooled, pi, Wg.T, bg[None, :], gg[None, :], gb[None, :], Wp.T,
        bp[None, :], pg[None, :], pb[None, :], Wf1.T[:512], Wf1.T[512:],
        bf1[None, :], f1g[None, :], f1b[None, :], Wf2.T, bf2[None, :],
        f2g[None, :], f2b[None, :])


# speedup vs baseline: 1.0084x; 1.0084x over previous
import functools

import jax
import jax.numpy as jnp
from jax import lax
from jax.experimental import pallas as pl
from jax.experimental.pallas import tpu as pltpu
from jax.experimental.pallas import tpu_sc as plsc

N = 10000
NPAD = 10240
NH = NPAD // 2
NB = 512
NBLK = NPAD // NB
D = 256
H = 128
G = 64

E = 160000
NC = 2
NS = 16
NT = NC * NS
TILE_E = 10240
EPAD = TILE_E * NS
ROWS_E = EPAD // 128
NCHUNK_E = TILE_E // 128
TILE_D = 5120
EPAD_D = TILE_D * NT
ROWS_D = EPAD_D // 128
NCHUNK_D = TILE_D // 128
NROWS_T = NH // NS

_mesh = plsc.VectorSubcoreMesh(core_axis_name="c", subcore_axis_name="s")
_sc_params = pltpu.CompilerParams(needs_layout_passes=False)


@functools.partial(
    pl.kernel,
    out_type=jax.ShapeDtypeStruct((NT, NPAD), jnp.float32),
    scratch_types=[
        pltpu.VMEM((NCHUNK_D, 128), jnp.int32),
        pltpu.VMEM((NPAD,), jnp.float32),
        pltpu.SemaphoreType.DMA,
    ],
    mesh=_mesh,
    compiler_params=_sc_params,
)
def _deg_kernel(dst_h, out_h, idx_v, deg_v, sem):
    c = lax.axis_index("c")
    s = lax.axis_index("s")
    tile = s * NC + c
    zero = jnp.zeros((16,), jnp.float32)

    def fill(i, _):
        deg_v[pl.ds(i * 16, 16)] = zero
        return 0

    lax.fori_loop(0, NPAD // 16, fill, 0)
    pltpu.sync_copy(dst_h.at[pl.ds(tile * NCHUNK_D, NCHUNK_D)], idx_v)
    ones = jnp.full((16,), 1.0, jnp.float32)

    def chunk(j, _):
        for k in range(8):
            iv = idx_v[j, pl.ds(k * 16, 16)]
            plsc.addupdate_scatter(deg_v, [iv], ones)
        return 0

    lax.fori_loop(0, NCHUNK_D, chunk, 0)
    pltpu.sync_copy(deg_v, out_h.at[tile])


def _make_scatter(p):
    @functools.partial(
        pl.kernel,
        out_type=jax.ShapeDtypeStruct((2 * NH, 128), jnp.float32),
        scratch_types=[
            pltpu.VMEM((NCHUNK_E, 128), jnp.int32),
            pltpu.VMEM((NCHUNK_E, 128), jnp.int32),
            pltpu.VMEM((4, 128, 128), jnp.float32),
            pltpu.VMEM_SHARED((NH, 128), jnp.float32),
            pltpu.SemaphoreType.DMA((4,)),
        ],
        mesh=_mesh,
        compiler_params=_sc_params,
    )
    def _scatter(hp_h, src_h, dst_h, out_h, idxs_v, idxd_v, rows_v, acc_s,
                 gsem):
        c = lax.axis_index("c")
        s = lax.axis_index("s")
        base = s * NROWS_T
        pltpu.sync_copy(hp_h.at[pl.ds(c * NPAD + p * NH + base, NROWS_T)],
                        acc_s.at[pl.ds(base, NROWS_T)])
        plsc.subcore_barrier()

        pltpu.sync_copy(src_h.at[pl.ds(c * ROWS_E + s * NCHUNK_E, NCHUNK_E)],
                        idxs_v)
        pltpu.sync_copy(dst_h.at[pl.ds(s * NCHUNK_E, NCHUNK_E)], idxd_v)

        def gather(j):
            b = lax.rem(j, 4)
            return pltpu.make_async_copy(
                hp_h.at[plsc.Indices(idxs_v.at[j], ignored_value=-1)],
                rows_v.at[b], gsem.at[b])

        gather(jnp.int32(0)).start()
        gather(jnp.int32(1)).start()
        gather(jnp.int32(2)).start()

        def chunk(j, _):
            b = lax.rem(j, 4)
            gather(j).wait()

            @pl.when(j < NCHUNK_E - 3)
            def _():
                gather(j + 3).start()

            pltpu.sync_copy(
                rows_v.at[b],
                acc_s.at[plsc.Indices(idxd_v.at[j], ignored_value=-1)],
                add=True)
            return 0

        lax.fori_loop(0, NCHUNK_E, chunk, 0)
        plsc.subcore_barrier()
        pltpu.sync_copy(acc_s.at[pl.ds(base, NROWS_T)],
                        out_h.at[pl.ds(c * NH + base, NROWS_T)])

    return _scatter


_scatter_lo = _make_scatter(0)
_scatter_hi = _make_scatter(1)


def _dinv128(degp_blk):
    deg = lax.dot_general(
        degp_blk, jnp.ones((NT, 128), jnp.float32),
        dimension_numbers=(((0,), (0,)), ((), ())),
        preferred_element_type=jnp.float32) + 1.0
    return lax.rsqrt(deg)


def _split_scaled(h, dinv):
    return jnp.stack([h[:, :128] * dinv, h[:, 128:] * dinv])


def _pre_body(x_ref, w_ref, degp_ref, hp_ref):
    dinv = _dinv128(degp_ref[...])
    h = jnp.dot(x_ref[...], w_ref[...], preferred_element_type=jnp.float32)
    hp_ref[...] = _split_scaled(h, dinv)


def _pre_call(xp, w1t, degp):
    return pl.pallas_call(
        _pre_body,
        grid=(NBLK,),
        in_specs=[
            pl.BlockSpec((NB, D), lambda nb: (nb, 0)),
            pl.BlockSpec((D, D), lambda nb: (0, 0)),
            pl.BlockSpec((NT, NB), lambda nb: (0, nb)),
        ],
        out_specs=pl.BlockSpec((2, NB, 128), lambda nb: (0, nb, 0)),
        out_shape=jax.ShapeDtypeStruct((2, NPAD, 128), jnp.float32),
    )(xp, w1t, degp)


def _seg_max(ho, bcol, lo, hi, p_ref):
    neg = jnp.float32(-jnp.inf)

    def seg(g, _):
        m = bcol == g.astype(jnp.float32)
        v = jnp.max(jnp.where(m, ho, neg), axis=0, keepdims=True)
        p_ref[pl.ds(g, 1), :] = jnp.maximum(p_ref[pl.ds(g, 1), :], v)
        return 0

    lax.fori_loop(lo, jnp.minimum(hi, G - 1) + 1, seg, 0)


def _update_body(acc_ref, degp_ref, b_ref, w_ref, bcol_ref, lo_ref, hi_ref,
                 hp_ref, p_ref):
    nb = pl.program_id(0)

    @pl.when(nb == 0)
    def _():
        p_ref[...] = jnp.full((G, D), -jnp.inf, jnp.float32)

    dinv = _dinv128(degp_ref[...])
    ho = jnp.concatenate([acc_ref[0] * dinv, acc_ref[1] * dinv], axis=-1)
    ho = jnp.maximum(ho + b_ref[...], 0.0)
    h = jnp.dot(ho, w_ref[...], preferred_element_type=jnp.float32)
    hp_ref[...] = _split_scaled(h, dinv)
    _seg_max(ho, bcol_ref[...], lo_ref[nb], hi_ref[nb], p_ref)


def _update_call(acc, degp, b, wt, bcol, lo, hi):
    return pl.pallas_call(
        _update_body,
        grid=(NBLK,),
        in_specs=[
            pl.BlockSpec((2, NB, 128), lambda nb: (0, nb, 0)),
            pl.BlockSpec((NT, NB), lambda nb: (0, nb)),
            pl.BlockSpec((1, D), lambda nb: (0, 0)),
            pl.BlockSpec((D, D), lambda nb: (0, 0)),
            pl.BlockSpec((NB, 1), lambda nb: (nb, 0)),
            pl.BlockSpec(memory_space=pltpu.MemorySpace.SMEM),
            pl.BlockSpec(memory_space=pltpu.MemorySpace.SMEM),
        ],
        out_specs=[
            pl.BlockSpec((2, NB, 128), lambda nb: (0, nb, 0)),
            pl.BlockSpec((G, D), lambda nb: (0, 0)),
        ],
        out_shape=[
            jax.ShapeDtypeStruct((2, NPAD, 128), jnp.float32),
            jax.ShapeDtypeStruct((G, D), jnp.float32),
        ],
    )(acc, degp, b, wt, bcol, lo, hi)


def _pool_body(acc_ref, degp_ref, b_ref, bcol_ref, lo_ref, hi_ref, p_ref):
    nb = pl.program_id(0)

    @pl.when(nb == 0)
    def _():
        p_ref[...] = jnp.full((G, D), -jnp.inf, jnp.float32)

    dinv = _dinv128(degp_ref[...])
    ho3 = jnp.concatenate([acc_ref[0] * dinv, acc_ref[1] * dinv], axis=-1)
    ho3 = jnp.maximum(ho3 + b_ref[...], 0.0)
    _seg_max(ho3, bcol_ref[...], lo_ref[nb], hi_ref[nb], p_ref)


def _pool_call(acc3, degp, b3, bcol, lo, hi):
    return pl.pallas_call(
        _pool_body,
        grid=(NBLK,),
        in_specs=[
            pl.BlockSpec((2, NB, 128), lambda nb: (0, nb, 0)),
            pl.BlockSpec((NT, NB), lambda nb: (0, nb)),
            pl.BlockSpec((1, D), lambda nb: (0, 0)),
            pl.BlockSpec((NB, 1), lambda nb: (nb, 0)),
            pl.BlockSpec(memory_space=pltpu.MemorySpace.SMEM),
            pl.BlockSpec(memory_space=pltpu.MemorySpace.SMEM),
        ],
        out_specs=pl.BlockSpec((G, D), lambda nb: (0, 0)),
        out_shape=jax.ShapeDtypeStruct((G, D), jnp.float32),
    )(acc3, degp, b3, bcol, lo, hi)


def _ln(x, g, b):
    m = jnp.mean(x, axis=-1, keepdims=True)
    v = jnp.mean((x - m) ** 2, axis=-1, keepdims=True)
    return (x - m) * lax.rsqrt(v + 1e-5) * g + b


def _head_body(pooled_ref, pi_ref, wgt_ref, bg_ref, gg_ref, gb_ref, wpt_ref,
               bp_ref, pg_ref, pb_ref, wf1a_ref, wf1b_ref, bf1_ref, f1g_ref,
               f1b_ref, wf2t_ref, bf2_ref, f2g_ref, f2b_ref, out_ref):
    ge = jnp.dot(pooled_ref[...], wgt_ref[...],
                 preferred_element_type=jnp.float32) + bg_ref[...]
    ge = jnp.maximum(_ln(ge, gg_ref[...], gb_ref[...]), 0.0)
    pe = jnp.dot(pi_ref[...], wpt_ref[...],
                 preferred_element_type=jnp.float32) + bp_ref[...]
    pe = jnp.maximum(_ln(pe, pg_ref[...], pb_ref[...]), 0.0)
    z = (jnp.dot(ge, wf1a_ref[...], preferred_element_type=jnp.float32)
         + jnp.dot(pe, wf1b_ref[...], preferred_element_type=jnp.float32)
         + bf1_ref[...])
    z = _ln(z, f1g_ref[...], f1b_ref[...])
    z = jnp.dot(z, wf2t_ref[...], preferred_element_type=jnp.float32) \
        + bf2_ref[...]
    out_ref[...] = _ln(z, f2g_ref[...], f2b_ref[...])


def _head_call(pooled, pi, *args):
    return pl.pallas_call(
        _head_body,
        out_shape=jax.ShapeDtypeStruct((G, 10), jnp.float32),
    )(pooled, pi, *args)


def _layer_scatter(hp, src, dst, dst_ok):
    hp = hp.reshape(2 * NPAD, 128)
    accs = []
    for p, call in ((0, _scatter_lo), (1, _scatter_hi)):
        keep = dst_ok & (dst // NH == p)
        srcq = jnp.where(keep[None, :], src[None, :]
                         + jnp.array([[0], [NPAD]], jnp.int32), -1)
        dstq = jnp.where(keep, dst - p * NH, -1)
        out = call(hp, srcq.reshape(2 * ROWS_E, 128),
                   dstq.reshape(ROWS_E, 128))
        accs.append(out.reshape(2, NH, 128))
    return jnp.concatenate(accs, axis=1)


def kernel(x, edge_index, batch, pi, W1, b1, W2, b2, W3, b3, Wg, bg, gg, gb,
           Wp, bp, pg, pb, Wf1, bf1, f1g, f1b, Wf2, bf2, f2g, f2b):
    f32 = jnp.float32
    src = edge_index[0]
    dst = edge_index[1]

    srcp = jnp.concatenate([src, jnp.zeros((EPAD - E,), jnp.int32)])
    dstp = jnp.concatenate([dst, jnp.zeros((EPAD - E,), jnp.int32)])
    dst_ok = jnp.arange(EPAD, dtype=jnp.int32) < E
    dstd = jnp.concatenate([dst, jnp.full((EPAD_D - E,), N, jnp.int32)])

    xp = jnp.pad(x, ((0, NPAD - N), (0, 0)))
    batchp = jnp.concatenate([batch, jnp.full((NPAD - N,), G, jnp.int32)])
    bcol = batchp.astype(f32)[:, None]
    lo = batchp[::NB]
    hi = batchp[NB - 1::NB]

    degp = _deg_kernel(dstd.reshape(ROWS_D, 128))

    hp1 = _pre_call(xp, W1.T, degp)
    acc1 = _layer_scatter(hp1, srcp, dstp, dst_ok)
    hp2, p1 = _update_call(acc1, degp, b1[None, :], W2.T, bcol, lo, hi)
    acc2 = _layer_scatter(hp2, srcp, dstp, dst_ok)
    hp3, p2 = _update_call(acc2, degp, b2[None, :], W3.T, bcol, lo, hi)
    acc3 = _layer_scatter(hp3, srcp, dstp, dst_ok)

    p3 = _pool_call(acc3, degp, b3[None, :], bcol, lo, hi)
    pooled = jnp.concatenate([p1, p2, p3], axis=1)

    return _head_call(
        pooled, pi, Wg.T, bg[None, :], gg[None, :], gb[None, :], Wp.T,
        bp[None, :], pg[None, :], pb[None, :], Wf1.T[:512], Wf1.T[512:],
        bf1[None, :], f1g[None, :], f1b[None, :], Wf2.T, bf2[None, :],
        f2g[None, :], f2b[None, :])


# speedup vs baseline: 1.0742x; 1.0652x over previous
import functools

import jax
import jax.numpy as jnp
from jax import lax
from jax.experimental import pallas as pl
from jax.experimental.pallas import tpu as pltpu
from jax.experimental.pallas import tpu_sc as plsc

N = 10000
NPAD = 10240
NH = NPAD // 2
NB = 512
NBLK = NPAD // NB
D = 256
H = 128
G = 64

E = 160000
NC = 2
NS = 16
NT = NC * NS
TILE_E = 10240
EPAD = TILE_E * NS
ROWS_E = EPAD // 128
NCHUNK_E = TILE_E // 128
TILE_D = 5120
EPAD_D = TILE_D * NT
ROWS_D = EPAD_D // 128
NCHUNK_D = TILE_D // 128
NROWS_T = NH // NS

_mesh = plsc.VectorSubcoreMesh(core_axis_name="c", subcore_axis_name="s")
_sc_params = pltpu.CompilerParams(needs_layout_passes=False)


@functools.partial(
    pl.kernel,
    out_type=jax.ShapeDtypeStruct((NT, NPAD), jnp.float32),
    scratch_types=[
        pltpu.VMEM((NCHUNK_D, 128), jnp.int32),
        pltpu.VMEM((NPAD,), jnp.float32),
        pltpu.SemaphoreType.DMA,
    ],
    mesh=_mesh,
    compiler_params=_sc_params,
)
def _deg_kernel(dst_h, out_h, idx_v, deg_v, sem):
    c = lax.axis_index("c")
    s = lax.axis_index("s")
    tile = s * NC + c
    zero = jnp.zeros((16,), jnp.float32)

    def fill(i, _):
        deg_v[pl.ds(i * 16, 16)] = zero
        return 0

    lax.fori_loop(0, NPAD // 16, fill, 0)
    pltpu.sync_copy(dst_h.at[pl.ds(tile * NCHUNK_D, NCHUNK_D)], idx_v)
    ones = jnp.full((16,), 1.0, jnp.float32)

    def chunk(j, _):
        for k in range(8):
            iv = idx_v[j, pl.ds(k * 16, 16)]
            plsc.addupdate_scatter(deg_v, [iv], ones)
        return 0

    lax.fori_loop(0, NCHUNK_D, chunk, 0)
    pltpu.sync_copy(deg_v, out_h.at[tile])


@functools.partial(
    pl.kernel,
    out_type=jax.ShapeDtypeStruct((2 * NPAD, 128), jnp.float32),
    scratch_types=[
        pltpu.VMEM((NCHUNK_E, 128), jnp.int32),
        pltpu.VMEM((NCHUNK_E, 128), jnp.int32),
        pltpu.VMEM((4, 128, 128), jnp.float32),
        pltpu.VMEM_SHARED((NH, 128), jnp.float32),
        pltpu.SemaphoreType.DMA((4,)),
    ],
    mesh=_mesh,
    compiler_params=_sc_params,
)
def _scatter(hp_h, src_h, dst_h, out_h, idxs_v, idxd_v, rows_v, acc_s, gsem):
    c = lax.axis_index("c")
    s = lax.axis_index("s")
    base = s * NROWS_T

    for p in (0, 1):
        pltpu.sync_copy(hp_h.at[pl.ds(c * NPAD + p * NH + base, NROWS_T)],
                        acc_s.at[pl.ds(base, NROWS_T)])
        plsc.subcore_barrier()

        pltpu.sync_copy(
            src_h.at[pl.ds((p * 2 + c) * ROWS_E + s * NCHUNK_E, NCHUNK_E)],
            idxs_v)
        pltpu.sync_copy(
            dst_h.at[pl.ds(p * ROWS_E + s * NCHUNK_E, NCHUNK_E)], idxd_v)

        def gather(j):
            b = lax.rem(j, 4)
            return pltpu.make_async_copy(
                hp_h.at[plsc.Indices(idxs_v.at[j], ignored_value=-1)],
                rows_v.at[b], gsem.at[b])

        gather(jnp.int32(0)).start()
        gather(jnp.int32(1)).start()
        gather(jnp.int32(2)).start()

        def chunk(j, _):
            b = lax.rem(j, 4)
            gather(j).wait()

            @pl.when(j < NCHUNK_E - 3)
            def _():
                gather(j + 3).start()

            pltpu.sync_copy(
                rows_v.at[b],
                acc_s.at[plsc.Indices(idxd_v.at[j], ignored_value=-1)],
                add=True)
            return 0

        lax.fori_loop(0, NCHUNK_E, chunk, 0)
        plsc.subcore_barrier()
        pltpu.sync_copy(acc_s.at[pl.ds(base, NROWS_T)],
                        out_h.at[pl.ds(c * NPAD + p * NH + base, NROWS_T)])
        plsc.subcore_barrier()


def _dinv128(degp_blk):
    deg = lax.dot_general(
        degp_blk, jnp.ones((NT, 128), jnp.float32),
        dimension_numbers=(((0,), (0,)), ((), ())),
        preferred_element_type=jnp.float32) + 1.0
    return lax.rsqrt(deg)


def _split_scaled(h, dinv):
    return jnp.stack([h[:, :128] * dinv, h[:, 128:] * dinv])


def _pre_body(x_ref, w_ref, degp_ref, hp_ref):
    dinv = _dinv128(degp_ref[...])
    h = jnp.dot(x_ref[...], w_ref[...], preferred_element_type=jnp.float32)
    hp_ref[...] = _split_scaled(h, dinv)


def _pre_call(xp, w1t, degp):
    return pl.pallas_call(
        _pre_body,
        grid=(NBLK,),
        in_specs=[
            pl.BlockSpec((NB, D), lambda nb: (nb, 0)),
            pl.BlockSpec((D, D), lambda nb: (0, 0)),
            pl.BlockSpec((NT, NB), lambda nb: (0, nb)),
        ],
        out_specs=pl.BlockSpec((2, NB, 128), lambda nb: (0, nb, 0)),
        out_shape=jax.ShapeDtypeStruct((2, NPAD, 128), jnp.float32),
    )(xp, w1t, degp)


def _seg_max(ho, bcol, lo, hi, p_ref):
    neg = jnp.float32(-jnp.inf)

    def seg(g, _):
        m = bcol == g.astype(jnp.float32)
        v = jnp.max(jnp.where(m, ho, neg), axis=0, keepdims=True)
        p_ref[pl.ds(g, 1), :] = jnp.maximum(p_ref[pl.ds(g, 1), :], v)
        return 0

    lax.fori_loop(lo, jnp.minimum(hi, G - 1) + 1, seg, 0)


def _update_body(acc_ref, degp_ref, b_ref, w_ref, bcol_ref, lo_ref, hi_ref,
                 hp_ref, p_ref):
    nb = pl.program_id(0)

    @pl.when(nb == 0)
    def _():
        p_ref[...] = jnp.full((G, D), -jnp.inf, jnp.float32)

    dinv = _dinv128(degp_ref[...])
    ho = jnp.concatenate([acc_ref[0] * dinv, acc_ref[1] * dinv], axis=-1)
    ho = jnp.maximum(ho + b_ref[...], 0.0)
    h = jnp.dot(ho, w_ref[...], preferred_element_type=jnp.float32)
    hp_ref[...] = _split_scaled(h, dinv)
    _seg_max(ho, bcol_ref[...], lo_ref[nb], hi_ref[nb], p_ref)


def _update_call(acc, degp, b, wt, bcol, lo, hi):
    return pl.pallas_call(
        _update_body,
        grid=(NBLK,),
        in_specs=[
            pl.BlockSpec((2, NB, 128), lambda nb: (0, nb, 0)),
            pl.BlockSpec((NT, NB), lambda nb: (0, nb)),
            pl.BlockSpec((1, D), lambda nb: (0, 0)),
            pl.BlockSpec((D, D), lambda nb: (0, 0)),
            pl.BlockSpec((NB, 1), lambda nb: (nb, 0)),
            pl.BlockSpec(memory_space=pltpu.MemorySpace.SMEM),
            pl.BlockSpec(memory_space=pltpu.MemorySpace.SMEM),
        ],
        out_specs=[
            pl.BlockSpec((2, NB, 128), lambda nb: (0, nb, 0)),
            pl.BlockSpec((G, D), lambda nb: (0, 0)),
        ],
        out_shape=[
            jax.ShapeDtypeStruct((2, NPAD, 128), jnp.float32),
            jax.ShapeDtypeStruct((G, D), jnp.float32),
        ],
    )(acc, degp, b, wt, bcol, lo, hi)


def _pool_body(acc_ref, degp_ref, b_ref, bcol_ref, lo_ref, hi_ref, p_ref):
    nb = pl.program_id(0)

    @pl.when(nb == 0)
    def _():
        p_ref[...] = jnp.full((G, D), -jnp.inf, jnp.float32)

    dinv = _dinv128(degp_ref[...])
    ho3 = jnp.concatenate([acc_ref[0] * dinv, acc_ref[1] * dinv], axis=-1)
    ho3 = jnp.maximum(ho3 + b_ref[...], 0.0)
    _seg_max(ho3, bcol_ref[...], lo_ref[nb], hi_ref[nb], p_ref)


def _pool_call(acc3, degp, b3, bcol, lo, hi):
    return pl.pallas_call(
        _pool_body,
        grid=(NBLK,),
        in_specs=[
            pl.BlockSpec((2, NB, 128), lambda nb: (0, nb, 0)),
            pl.BlockSpec((NT, NB), lambda nb: (0, nb)),
            pl.BlockSpec((1, D), lambda nb: (0, 0)),
            pl.BlockSpec((NB, 1), lambda nb: (nb, 0)),
            pl.BlockSpec(memory_space=pltpu.MemorySpace.SMEM),
            pl.BlockSpec(memory_space=pltpu.MemorySpace.SMEM),
        ],
        out_specs=pl.BlockSpec((G, D), lambda nb: (0, 0)),
        out_shape=jax.ShapeDtypeStruct((G, D), jnp.float32),
    )(acc3, degp, b3, bcol, lo, hi)


def _ln(x, g, b):
    m = jnp.mean(x, axis=-1, keepdims=True)
    v = jnp.mean((x - m) ** 2, axis=-1, keepdims=True)
    return (x - m) * lax.rsqrt(v + 1e-5) * g + b


def _head_body(pooled_ref, pi_ref, wgt_ref, bg_ref, gg_ref, gb_ref, wpt_ref,
               bp_ref, pg_ref, pb_ref, wf1a_ref, wf1b_ref, bf1_ref, f1g_ref,
               f1b_ref, wf2t_ref, bf2_ref, f2g_ref, f2b_ref, out_ref):
    ge = jnp.dot(pooled_ref[...], wgt_ref[...],
                 preferred_element_type=jnp.float32) + bg_ref[...]
    ge = jnp.maximum(_ln(ge, gg_ref[...], gb_ref[...]), 0.0)
    pe = jnp.dot(pi_ref[...], wpt_ref[...],
                 preferred_element_type=jnp.float32) + bp_ref[...]
    pe = jnp.maximum(_ln(pe, pg_ref[...], pb_ref[...]), 0.0)
    z = (jnp.dot(ge, wf1a_ref[...], preferred_element_type=jnp.float32)
         + jnp.dot(pe, wf1b_ref[...], preferred_element_type=jnp.float32)
         + bf1_ref[...])
    z = _ln(z, f1g_ref[...], f1b_ref[...])
    z = jnp.dot(z, wf2t_ref[...], preferred_element_type=jnp.float32) \
        + bf2_ref[...]
    out_ref[...] = _ln(z, f2g_ref[...], f2b_ref[...])


def _head_call(pooled, pi, *args):
    return pl.pallas_call(
        _head_body,
        out_shape=jax.ShapeDtypeStruct((G, 10), jnp.float32),
    )(pooled, pi, *args)


def _edge_indices(src, dst, dst_ok):
    srcqs, dstqs = [], []
    for p in (0, 1):
        keep = dst_ok & (dst // NH == p)
        srcqs.append(jnp.where(keep[None, :], src[None, :]
                               + jnp.array([[0], [NPAD]], jnp.int32), -1))
        dstqs.append(jnp.where(keep, dst - p * NH, -1))
    return (jnp.concatenate(srcqs).reshape(4 * ROWS_E, 128),
            jnp.concatenate(dstqs).reshape(2 * ROWS_E, 128))


def _layer_scatter(hp, srcq, dstq):
    return _scatter(hp.reshape(2 * NPAD, 128), srcq, dstq) \
        .reshape(2, NPAD, 128)


def kernel(x, edge_index, batch, pi, W1, b1, W2, b2, W3, b3, Wg, bg, gg, gb,
           Wp, bp, pg, pb, Wf1, bf1, f1g, f1b, Wf2, bf2, f2g, f2b):
    f32 = jnp.float32
    src = edge_index[0]
    dst = edge_index[1]

    srcp = jnp.concatenate([src, jnp.zeros((EPAD - E,), jnp.int32)])
    dstp = jnp.concatenate([dst, jnp.zeros((EPAD - E,), jnp.int32)])
    dst_ok = jnp.arange(EPAD, dtype=jnp.int32) < E
    dstd = jnp.concatenate([dst, jnp.full((EPAD_D - E,), N, jnp.int32)])

    xp = jnp.pad(x, ((0, NPAD - N), (0, 0)))
    batchp = jnp.concatenate([batch, jnp.full((NPAD - N,), G, jnp.int32)])
    bcol = batchp.astype(f32)[:, None]
    lo = batchp[::NB]
    hi = batchp[NB - 1::NB]

    degp = _deg_kernel(dstd.reshape(ROWS_D, 128))

    srcq, dstq = _edge_indices(srcp, dstp, dst_ok)
    hp1 = _pre_call(xp, W1.T, degp)
    acc1 = _layer_scatter(hp1, srcq, dstq)
    hp2, p1 = _update_call(acc1, degp, b1[None, :], W2.T, bcol, lo, hi)
    acc2 = _layer_scatter(hp2, srcq, dstq)
    hp3, p2 = _update_call(acc2, degp, b2[None, :], W3.T, bcol, lo, hi)
    acc3 = _layer_scatter(hp3, srcq, dstq)

    p3 = _pool_call(acc3, degp, b3[None, :], bcol, lo, hi)
    pooled = jnp.concatenate([p1, p2, p3], axis=1)

    return _head_call(
        pooled, pi, Wg.T, bg[None, :], gg[None, :], gb[None, :], Wp.T,
        bp[None, :], pg[None, :], pb[None, :], Wf1.T[:512], Wf1.T[512:],
        bf1[None, :], f1g[None, :], f1b[None, :], Wf2.T, bf2[None, :],
        f2g[None, :], f2b[None, :])
